# Initial kernel scaffold; baseline (speedup 1.0000x reference)
import jax
import jax.numpy as jnp
from jax.experimental import pallas as pl


def kernel(x, mask_token, noise):
    raise NotImplementedError("write your pallas kernel here")


# speedup vs baseline: 2.6546x; 2.6546x over previous
import functools

import jax
import jax.numpy as jnp
from jax.experimental import pallas as pl
from jax.experimental.pallas import tpu as pltpu

MASK_RATIO = 0.6


def _mask_kernel(noise_ref, mask_ref, *, k):
    noise = noise_ref[...]
    b, m = noise.shape
    bits = jax.lax.bitcast_convert_type(noise, jnp.int32)

    v = jnp.zeros((b, 1), jnp.int32)
    for bit in range(29, -1, -1):
        cand = v + (1 << bit)
        cnt = jnp.sum((bits < cand).astype(jnp.float32), axis=1, keepdims=True)
        v = jnp.where(cnt < k, cand, v)
    c_less = jnp.sum((bits < v).astype(jnp.float32), axis=1, keepdims=True)

    eq = (bits == v).astype(jnp.float32)
    sub = 128
    rows = m // sub
    e3 = eq.reshape(b, rows, sub)
    i0 = jax.lax.broadcasted_iota(jnp.int32, (sub, sub), 0)
    i1 = jax.lax.broadcasted_iota(jnp.int32, (sub, sub), 1)
    tri_s = (i0 < i1).astype(jnp.float32)
    inner = jax.lax.dot_general(
        e3, tri_s, (((2,), (0,)), ((), ())),
        preferred_element_type=jnp.float32)
    rowtot = jnp.sum(e3, axis=2)
    j0 = jax.lax.broadcasted_iota(jnp.int32, (rows, rows), 0)
    j1 = jax.lax.broadcasted_iota(jnp.int32, (rows, rows), 1)
    tri_r = (j0 < j1).astype(jnp.float32)
    rowexcl = jax.lax.dot_general(
        rowtot, tri_r, (((1,), (0,)), ((), ())),
        preferred_element_type=jnp.float32)
    pre = (inner + rowexcl[:, :, None]).reshape(b, m)

    quota = k - c_less
    masked = (bits < v) | ((eq > 0.0) & (pre < quota))
    mask_ref[...] = masked.astype(jnp.float32)


def _apply_kernel(x_ref, mask_ref, tok_ref, out_ref, *, bm):
    i = pl.program_id(0)
    j = pl.program_id(1)
    m = mask_ref[i, pl.ds(j * bm, bm)]
    x = x_ref[0]
    t = tok_ref[0, 0]
    sel = m[:, None] > 0.5
    out_ref[0] = jnp.where(sel, t[None, :], x)


@jax.jit
def kernel(x, mask_token, noise):
    b, m, c = x.shape
    k = int(m * MASK_RATIO)

    mask_bool = pl.pallas_call(
        functools.partial(_mask_kernel, k=k),
        out_shape=jax.ShapeDtypeStruct((b, m), jnp.float32),
    )(noise)

    bm = 1024
    grid = (b, m // bm)
    x_masked = pl.pallas_call(
        functools.partial(_apply_kernel, bm=bm),
        grid=grid,
        in_specs=[
            pl.BlockSpec((1, bm, c), lambda i, j: (i, j, 0)),
            pl.BlockSpec((b, m), lambda i, j: (0, 0)),
            pl.BlockSpec((1, 1, c), lambda i, j: (0, 0, 0)),
        ],
        out_specs=pl.BlockSpec((1, bm, c), lambda i, j: (i, j, 0)),
        out_shape=jax.ShapeDtypeStruct((b, m, c), x.dtype),
        compiler_params=pltpu.CompilerParams(
            dimension_semantics=("parallel", "arbitrary"),
        ),
    )(x, mask_bool, mask_token)

    return (x_masked, mask_bool)


# speedup vs baseline: 2.7560x; 1.0382x over previous
import functools

import jax
import jax.numpy as jnp
from jax.experimental import pallas as pl
from jax.experimental.pallas import tpu as pltpu

MASK_RATIO = 0.6


def _mask_kernel(noise_ref, mask_ref, *, k):
    noise = noise_ref[...]
    b, m = noise.shape
    bits = jax.lax.bitcast_convert_type(noise, jnp.int32)

    v = jnp.zeros((b, 1), jnp.int32)
    for bit in range(29, -1, -1):
        cand = v + (1 << bit)
        cnt = jnp.sum((bits < cand).astype(jnp.float32), axis=1, keepdims=True)
        v = jnp.where(cnt < k, cand, v)
    c_less = jnp.sum((bits < v).astype(jnp.float32), axis=1, keepdims=True)

    eq = (bits == v).astype(jnp.float32)
    sub = 128
    rows = m // sub
    e3 = eq.reshape(b, rows, sub)
    i0 = jax.lax.broadcasted_iota(jnp.int32, (sub, sub), 0)
    i1 = jax.lax.broadcasted_iota(jnp.int32, (sub, sub), 1)
    tri_s = (i0 < i1).astype(jnp.float32)
    inner = jax.lax.dot_general(
        e3, tri_s, (((2,), (0,)), ((), ())),
        preferred_element_type=jnp.float32)
    rowtot = jnp.sum(e3, axis=2)
    j0 = jax.lax.broadcasted_iota(jnp.int32, (rows, rows), 0)
    j1 = jax.lax.broadcasted_iota(jnp.int32, (rows, rows), 1)
    tri_r = (j0 < j1).astype(jnp.float32)
    rowexcl = jax.lax.dot_general(
        rowtot, tri_r, (((1,), (0,)), ((), ())),
        preferred_element_type=jnp.float32)
    pre = (inner + rowexcl[:, :, None]).reshape(b, m)

    quota = k - c_less
    masked = (bits < v) | ((eq > 0.0) & (pre < quota))
    mask_ref[...] = masked.astype(jnp.float32)


def _apply_kernel(x_ref, mask_ref, tok_ref, out_ref, *, bm):
    i = pl.program_id(0)
    j = pl.program_id(1)
    m = mask_ref[i, pl.ds(j * bm, bm)]
    x = x_ref[0]
    t = tok_ref[0, 0]
    sel = m[:, None] > 0.5
    out_ref[0] = jnp.where(sel, t[None, :], x)


@jax.jit
def kernel(x, mask_token, noise):
    b, m, c = x.shape
    k = int(m * MASK_RATIO)

    mask_bool = pl.pallas_call(
        functools.partial(_mask_kernel, k=k),
        out_shape=jax.ShapeDtypeStruct((b, m), jnp.float32),
    )(noise)

    bm = 2048
    grid = (b, m // bm)
    x_masked = pl.pallas_call(
        functools.partial(_apply_kernel, bm=bm),
        grid=grid,
        in_specs=[
            pl.BlockSpec((1, bm, c), lambda i, j: (i, j, 0)),
            pl.BlockSpec((b, m), lambda i, j: (0, 0)),
            pl.BlockSpec((1, 1, c), lambda i, j: (0, 0, 0)),
        ],
        out_specs=pl.BlockSpec((1, bm, c), lambda i, j: (i, j, 0)),
        out_shape=jax.ShapeDtypeStruct((b, m, c), x.dtype),
        compiler_params=pltpu.CompilerParams(
            dimension_semantics=("parallel", "arbitrary"),
        ),
    )(x, mask_bool, mask_token)

    return (x_masked, mask_bool)


# speedup vs baseline: 2.7987x; 1.0155x over previous
import functools

import jax
import jax.numpy as jnp
from jax.experimental import pallas as pl
from jax.experimental.pallas import tpu as pltpu

MASK_RATIO = 0.6


def _mask_kernel(noise_ref, mask_ref, *, k):
    noise = noise_ref[...]
    b, m = noise.shape
    bits = jax.lax.bitcast_convert_type(noise, jnp.int32)

    v = jnp.zeros((b, 1), jnp.int32)
    for bit in range(29, -1, -1):
        cand = v + (1 << bit)
        cnt = jnp.sum((bits < cand).astype(jnp.float32), axis=1, keepdims=True)
        v = jnp.where(cnt < k, cand, v)
    c_less = jnp.sum((bits < v).astype(jnp.float32), axis=1, keepdims=True)

    eq = (bits == v).astype(jnp.float32)
    sub = 128
    rows = m // sub
    e3 = eq.reshape(b, rows, sub)
    i0 = jax.lax.broadcasted_iota(jnp.int32, (sub, sub), 0)
    i1 = jax.lax.broadcasted_iota(jnp.int32, (sub, sub), 1)
    tri_s = (i0 < i1).astype(jnp.float32)
    inner = jax.lax.dot_general(
        e3, tri_s, (((2,), (0,)), ((), ())),
        preferred_element_type=jnp.float32)
    rowtot = jnp.sum(e3, axis=2)
    j0 = jax.lax.broadcasted_iota(jnp.int32, (rows, rows), 0)
    j1 = jax.lax.broadcasted_iota(jnp.int32, (rows, rows), 1)
    tri_r = (j0 < j1).astype(jnp.float32)
    rowexcl = jax.lax.dot_general(
        rowtot, tri_r, (((1,), (0,)), ((), ())),
        preferred_element_type=jnp.float32)
    pre = (inner + rowexcl[:, :, None]).reshape(b, m)

    quota = k - c_less
    masked = (bits < v) | ((eq > 0.0) & (pre < quota))
    mask_ref[...] = masked.astype(jnp.float32)


def _apply_kernel(x_ref, mask_ref, tok_ref, out_ref, *, bm):
    i = pl.program_id(0)
    j = pl.program_id(1)
    m = mask_ref[i, pl.ds(j * bm, bm)]
    x = x_ref[0]
    t = tok_ref[0, 0]
    sel = m[:, None] > 0.5
    out_ref[0] = jnp.where(sel, t[None, :], x)


@jax.jit
def kernel(x, mask_token, noise):
    b, m, c = x.shape
    k = int(m * MASK_RATIO)

    mask_bool = pl.pallas_call(
        functools.partial(_mask_kernel, k=k),
        out_shape=jax.ShapeDtypeStruct((b, m), jnp.float32),
    )(noise)

    bm = 4096
    grid = (b, m // bm)
    x_masked = pl.pallas_call(
        functools.partial(_apply_kernel, bm=bm),
        grid=grid,
        in_specs=[
            pl.BlockSpec((1, bm, c), lambda i, j: (i, j, 0)),
            pl.BlockSpec((b, m), lambda i, j: (0, 0)),
            pl.BlockSpec((1, 1, c), lambda i, j: (0, 0, 0)),
        ],
        out_specs=pl.BlockSpec((1, bm, c), lambda i, j: (i, j, 0)),
        out_shape=jax.ShapeDtypeStruct((b, m, c), x.dtype),
        compiler_params=pltpu.CompilerParams(
            dimension_semantics=("parallel", "arbitrary"),
        ),
    )(x, mask_bool, mask_token)

    return (x_masked, mask_bool)


# speedup vs baseline: 2.9151x; 1.0416x over previous
import functools

import jax
import jax.numpy as jnp
from jax.experimental import pallas as pl
from jax.experimental.pallas import tpu as pltpu

MASK_RATIO = 0.6


def _build_mask(noise, k):
    b, m = noise.shape
    bits = jax.lax.bitcast_convert_type(noise, jnp.int32)

    v = jnp.zeros((b, 1), jnp.int32)
    for bit in range(29, -1, -1):
        cand = v + (1 << bit)
        cnt = jnp.sum((bits < cand).astype(jnp.float32), axis=1, keepdims=True)
        v = jnp.where(cnt < k, cand, v)
    c_less = jnp.sum((bits < v).astype(jnp.float32), axis=1, keepdims=True)

    eq = (bits == v).astype(jnp.float32)
    sub = 128
    rows = m // sub
    e3 = eq.reshape(b, rows, sub)
    i0 = jax.lax.broadcasted_iota(jnp.int32, (sub, sub), 0)
    i1 = jax.lax.broadcasted_iota(jnp.int32, (sub, sub), 1)
    tri_s = (i0 < i1).astype(jnp.float32)
    inner = jax.lax.dot_general(
        e3, tri_s, (((2,), (0,)), ((), ())),
        preferred_element_type=jnp.float32)
    rowtot = jnp.sum(e3, axis=2)
    j0 = jax.lax.broadcasted_iota(jnp.int32, (rows, rows), 0)
    j1 = jax.lax.broadcasted_iota(jnp.int32, (rows, rows), 1)
    tri_r = (j0 < j1).astype(jnp.float32)
    rowexcl = jax.lax.dot_general(
        rowtot, tri_r, (((1,), (0,)), ((), ())),
        preferred_element_type=jnp.float32)
    pre = (inner + rowexcl[:, :, None]).reshape(b, m)

    quota = k - c_less
    masked = (bits < v) | ((eq > 0.0) & (pre < quota))
    return masked.astype(jnp.float32)


def _fused_kernel(noise_ref, x_ref, tok_ref, out_ref, mask_ref, *, k, bm):
    i = pl.program_id(0)
    j = pl.program_id(1)

    @pl.when((i == 0) & (j == 0))
    def _():
        mask_ref[...] = _build_mask(noise_ref[...], k)

    m = mask_ref[i, pl.ds(j * bm, bm)]
    sel = m[:, None] > 0.5
    out_ref[0] = jnp.where(sel, tok_ref[0, 0][None, :], x_ref[0])


@jax.jit
def kernel(x, mask_token, noise):
    b, m, c = x.shape
    k = int(m * MASK_RATIO)
    bm = 4096
    grid = (b, m // bm)

    x_masked, mask_bool = pl.pallas_call(
        functools.partial(_fused_kernel, k=k, bm=bm),
        grid=grid,
        in_specs=[
            pl.BlockSpec((b, m), lambda i, j: (0, 0)),
            pl.BlockSpec((1, bm, c), lambda i, j: (i, j, 0)),
            pl.BlockSpec((1, 1, c), lambda i, j: (0, 0, 0)),
        ],
        out_specs=[
            pl.BlockSpec((1, bm, c), lambda i, j: (i, j, 0)),
            pl.BlockSpec((b, m), lambda i, j: (0, 0)),
        ],
        out_shape=[
            jax.ShapeDtypeStruct((b, m, c), x.dtype),
            jax.ShapeDtypeStruct((b, m), jnp.float32),
        ],
        compiler_params=pltpu.CompilerParams(
            dimension_semantics=("arbitrary", "arbitrary"),
        ),
    )(noise, x, mask_token)

    return (x_masked, mask_bool)


# speedup vs baseline: 2.9379x; 1.0078x over previous
import functools

import jax
import jax.numpy as jnp
from jax.experimental import pallas as pl
from jax.experimental.pallas import tpu as pltpu

MASK_RATIO = 0.6


def _build_mask(noise, k):
    b, m = noise.shape
    bits = jax.lax.bitcast_convert_type(noise, jnp.int32)

    v = jnp.zeros((b, 1), jnp.int32)
    for bit in range(29, -1, -1):
        cand = v + (1 << bit)
        cnt = jnp.sum((bits < cand).astype(jnp.float32), axis=1, keepdims=True)
        v = jnp.where(cnt < k, cand, v)
    c_less = jnp.sum((bits < v).astype(jnp.float32), axis=1, keepdims=True)

    eq = (bits == v).astype(jnp.float32)
    sub = 128
    rows = m // sub
    e3 = eq.reshape(b, rows, sub)
    i0 = jax.lax.broadcasted_iota(jnp.int32, (sub, sub), 0)
    i1 = jax.lax.broadcasted_iota(jnp.int32, (sub, sub), 1)
    tri_s = (i0 < i1).astype(jnp.float32)
    inner = jax.lax.dot_general(
        e3, tri_s, (((2,), (0,)), ((), ())),
        preferred_element_type=jnp.float32)
    rowtot = jnp.sum(e3, axis=2)
    j0 = jax.lax.broadcasted_iota(jnp.int32, (rows, rows), 0)
    j1 = jax.lax.broadcasted_iota(jnp.int32, (rows, rows), 1)
    tri_r = (j0 < j1).astype(jnp.float32)
    rowexcl = jax.lax.dot_general(
        rowtot, tri_r, (((1,), (0,)), ((), ())),
        preferred_element_type=jnp.float32)
    pre = (inner + rowexcl[:, :, None]).reshape(b, m)

    quota = k - c_less
    masked = (bits < v) | ((eq > 0.0) & (pre < quota))
    return masked.astype(jnp.float32)


def _fused_kernel(noise_ref, x_ref, tok_ref, out_ref, mask_ref, *, k, bm):
    i = pl.program_id(0)
    j = pl.program_id(1)

    @pl.when(j == 0)
    def _():
        row = noise_ref[pl.ds(i, 1), :]
        mask_ref[pl.ds(i, 1), :] = _build_mask(row, k)

    m = mask_ref[i, pl.ds(j * bm, bm)]
    sel = m[:, None] > 0.5
    out_ref[0] = jnp.where(sel, tok_ref[0, 0][None, :], x_ref[0])


@jax.jit
def kernel(x, mask_token, noise):
    b, m, c = x.shape
    k = int(m * MASK_RATIO)
    bm = 4096
    grid = (b, m // bm)

    x_masked, mask_bool = pl.pallas_call(
        functools.partial(_fused_kernel, k=k, bm=bm),
        grid=grid,
        in_specs=[
            pl.BlockSpec((b, m), lambda i, j: (0, 0)),
            pl.BlockSpec((1, bm, c), lambda i, j: (i, j, 0)),
            pl.BlockSpec((1, 1, c), lambda i, j: (0, 0, 0)),
        ],
        out_specs=[
            pl.BlockSpec((1, bm, c), lambda i, j: (i, j, 0)),
            pl.BlockSpec((b, m), lambda i, j: (0, 0)),
        ],
        out_shape=[
            jax.ShapeDtypeStruct((b, m, c), x.dtype),
            jax.ShapeDtypeStruct((b, m), jnp.float32),
        ],
        compiler_params=pltpu.CompilerParams(
            dimension_semantics=("arbitrary", "arbitrary"),
        ),
    )(noise, x, mask_token)

    return (x_masked, mask_bool)


# speedup vs baseline: 2.9813x; 1.0148x over previous
import functools

import jax
import jax.numpy as jnp
from jax.experimental import pallas as pl
from jax.experimental.pallas import tpu as pltpu

MASK_RATIO = 0.6


def _build_mask(noise, k):
    m = noise.shape[1]
    sub = 128
    rows = m // sub
    bits = jax.lax.bitcast_convert_type(noise, jnp.int32).reshape(rows, sub)

    v = jnp.int32(0)
    c_less = jnp.float32(0.0)
    for bit in range(29, -1, -1):
        cand = v + (1 << bit)
        cnt = jnp.sum((bits < cand).astype(jnp.float32))
        take = cnt < k
        v = jnp.where(take, cand, v)
        c_less = jnp.where(take, cnt, c_less)

    eq = (bits == v).astype(jnp.float32)
    i0 = jax.lax.broadcasted_iota(jnp.int32, (sub, sub), 0)
    i1 = jax.lax.broadcasted_iota(jnp.int32, (sub, sub), 1)
    tri_s = (i0 < i1).astype(jnp.float32)
    inner = jax.lax.dot_general(
        eq, tri_s, (((1,), (0,)), ((), ())),
        preferred_element_type=jnp.float32)
    rowtot = jnp.sum(eq, axis=1)[None, :]
    j0 = jax.lax.broadcasted_iota(jnp.int32, (rows, rows), 0)
    j1 = jax.lax.broadcasted_iota(jnp.int32, (rows, rows), 1)
    tri_r = (j0 < j1).astype(jnp.float32)
    rowexcl = jax.lax.dot_general(
        rowtot, tri_r, (((1,), (0,)), ((), ())),
        preferred_element_type=jnp.float32)
    pre = inner + rowexcl.reshape(rows, 1)

    quota = k - c_less
    masked = (bits < v) | ((eq > 0.0) & (pre < quota))
    return masked.astype(jnp.float32).reshape(1, m)


def _fused_kernel(noise_ref, x_ref, tok_ref, out_ref, mask_ref, *, k, bm):
    i = pl.program_id(0)
    j = pl.program_id(1)

    @pl.when(j == 0)
    def _():
        row = noise_ref[pl.ds(i, 1), :]
        mask_ref[pl.ds(i, 1), :] = _build_mask(row, k)

    m = mask_ref[i, pl.ds(j * bm, bm)]
    sel = m[:, None] > 0.5
    out_ref[0] = jnp.where(sel, tok_ref[0, 0][None, :], x_ref[0])


@jax.jit
def kernel(x, mask_token, noise):
    b, m, c = x.shape
    k = int(m * MASK_RATIO)
    bm = 4096
    grid = (b, m // bm)

    x_masked, mask_bool = pl.pallas_call(
        functools.partial(_fused_kernel, k=k, bm=bm),
        grid=grid,
        in_specs=[
            pl.BlockSpec((b, m), lambda i, j: (0, 0)),
            pl.BlockSpec((1, bm, c), lambda i, j: (i, j, 0)),
            pl.BlockSpec((1, 1, c), lambda i, j: (0, 0, 0)),
        ],
        out_specs=[
            pl.BlockSpec((1, bm, c), lambda i, j: (i, j, 0)),
            pl.BlockSpec((b, m), lambda i, j: (0, 0)),
        ],
        out_shape=[
            jax.ShapeDtypeStruct((b, m, c), x.dtype),
            jax.ShapeDtypeStruct((b, m), jnp.float32),
        ],
        compiler_params=pltpu.CompilerParams(
            dimension_semantics=("arbitrary", "arbitrary"),
        ),
    )(noise, x, mask_token)

    return (x_masked, mask_bool)
